# Initial kernel scaffold; baseline (speedup 1.0000x reference)
import jax
import jax.numpy as jnp
from jax.experimental import pallas as pl


def kernel(inputs, kernel, training):
    raise NotImplementedError("write your pallas kernel here")


# speedup vs baseline: 1.1454x; 1.1454x over previous
import jax
import jax.numpy as jnp
from jax.experimental import pallas as pl

FIELD = 26
D = 64
PAIRS = FIELD * (FIELD - 1) // 2
BT = 256


def _body(x_ref, k_ref, o_ref):
    x = x_ref[...]
    outs = []
    off = 0
    for i in range(FIELD - 1):
        cnt = FIELD - 1 - i
        xi = x[:, i * D:(i + 1) * D]
        xj = x[:, (i + 1) * D:]
        kk = k_ref[0, off:off + cnt * D]
        xi_t = jnp.tile(xi, (1, cnt))
        t = xi_t * xj * kk[None, :]
        tr = jnp.sum(t.reshape(BT, cnt, D), axis=-1)
        outs.append(tr)
        off += cnt * D
    o_ref[...] = jnp.concatenate(outs, axis=-1)


def kernel(inputs, kernel, training=False):
    b = inputs.shape[0]
    x2 = inputs.reshape(b, FIELD * D)
    kflat = kernel.reshape(1, PAIRS * D)
    grid = (b // BT,)
    out = pl.pallas_call(
        _body,
        grid=grid,
        in_specs=[
            pl.BlockSpec((BT, FIELD * D), lambda g: (g, 0)),
            pl.BlockSpec((1, PAIRS * D), lambda g: (0, 0)),
        ],
        out_specs=pl.BlockSpec((BT, PAIRS), lambda g: (g, 0)),
        out_shape=jax.ShapeDtypeStruct((b, PAIRS), jnp.float32),
    )(x2, kflat)
    return out
